# baseline (speedup 1.0000x reference)
import functools

import jax
import jax.numpy as jnp
from jax import lax
from jax.experimental import pallas as pl
from jax.experimental.pallas import tpu as pltpu
from jax.experimental.pallas import tpu_sc as plsc

LANES = 16
NC, NS = 2, 16
NW = NC * NS
C = 40
ROW_TILE = 4096
V_TILE = 4096
V_TILE_A = 6144


PACKW = 256


def _pack_halves(y):
    h = y.shape[1] // 2
    lo = lax.bitcast_convert_type(y[:, :h].astype(jnp.bfloat16), jnp.uint16)
    hi = lax.bitcast_convert_type(y[:, h:].astype(jnp.bfloat16), jnp.uint16)
    word = lax.bitcast_convert_type(
        lo.astype(jnp.uint32) | (hi.astype(jnp.uint32) << 16), jnp.int32)
    pad = jnp.zeros((y.shape[0], PACKW - h), jnp.int32)
    return jnp.concatenate([word, pad], axis=1)


def _proj_body(tok_ref, pth_ref, w1a_ref, w1b_ref, w1c_ref, b1_ref,
               tt1_ref, tt2_ref, pp_ref):
    t = tok_ref[...]
    tt1_ref[...] = _pack_halves(
        jnp.dot(t, w1a_ref[...], preferred_element_type=jnp.float32) + b1_ref[...])
    tt2_ref[...] = _pack_halves(
        jnp.dot(t, w1c_ref[...], preferred_element_type=jnp.float32))
    pp_ref[...] = _pack_halves(
        jnp.dot(pth_ref[...], w1b_ref[...], preferred_element_type=jnp.float32))


def _project_tables(token_table, path_table, W1, b1):
    VT, DT = token_table.shape
    VP, DP = path_table.shape
    DC = W1.shape[0]
    W1a, W1b, W1c = W1[:DT], W1[DT:DT + DP], W1[DT + DP:]
    grid = (pl.cdiv(max(VT, VP), ROW_TILE),)
    return pl.pallas_call(
        _proj_body,
        grid=grid,
        in_specs=[
            pl.BlockSpec((ROW_TILE, DT), lambda i: (i, 0)),
            pl.BlockSpec((ROW_TILE, DP), lambda i: (i, 0)),
            pl.BlockSpec((DT, DC), lambda i: (0, 0)),
            pl.BlockSpec((DP, DC), lambda i: (0, 0)),
            pl.BlockSpec((DT, DC), lambda i: (0, 0)),
            pl.BlockSpec((1, DC), lambda i: (0, 0)),
        ],
        out_specs=[
            pl.BlockSpec((ROW_TILE, PACKW), lambda i: (i, 0)),
            pl.BlockSpec((ROW_TILE, PACKW), lambda i: (i, 0)),
            pl.BlockSpec((ROW_TILE, PACKW), lambda i: (i, 0)),
        ],
        out_shape=[
            jax.ShapeDtypeStruct((VT, PACKW), jnp.int32),
            jax.ShapeDtypeStruct((VT, PACKW), jnp.int32),
            jax.ShapeDtypeStruct((VP, PACKW), jnp.int32),
        ],
    )(token_table, path_table, W1a, W1b, W1c, b1.reshape(1, DC))


def _make_sc_combine(B, L, DC):
    nchunk = L // C
    b_per_w = B // NW
    idx_per_chunk = 3 * C
    DH = DC // 2
    nsl = DH // LANES
    mesh = plsc.VectorSubcoreMesh(core_axis_name="c", subcore_axis_name="s")

    @functools.partial(
        pl.kernel,
        out_type=jax.ShapeDtypeStruct((B, DC), jnp.float32),
        mesh=mesh,
        scratch_types=[
            pltpu.VMEM((b_per_w * nchunk, idx_per_chunk), jnp.int32),
            pltpu.VMEM((3, C, PACKW), jnp.int32),
            pltpu.VMEM((3, C, PACKW), jnp.int32),
            pltpu.VMEM((3, C, PACKW), jnp.int32),
            pltpu.VMEM((DC,), jnp.float32),
            pltpu.SemaphoreType.DMA,
            pltpu.SemaphoreType.DMA,
            pltpu.SemaphoreType.DMA,
        ],
    )
    def sc_combine(idx_hbm, tt1_hbm, pp_hbm, tt2_hbm, cv_hbm,
                   idxall, g0, g1, g2, accb, sem0, sem1, sem2):
        wid = lax.axis_index("s") * NC + lax.axis_index("c")
        tables = (tt1_hbm, pp_hbm, tt2_hbm)
        bufs = (g0, g1, g2)
        sems = (sem0, sem1, sem2)
        nrows = b_per_w * nchunk
        pltpu.sync_copy(idx_hbm.at[pl.ds(wid * nrows, nrows)], idxall)

        def b_body(b, carry):
            row = wid * b_per_w + b

            def fire(c):
                par = c % 3
                handles = []
                for k in range(3):
                    cp = pltpu.make_async_copy(
                        tables[k].at[idxall.at[b * nchunk + c, pl.ds(k * C, C)]],
                        bufs[k].at[par], sems[par])
                    cp.start()
                    handles.append(cp)
                return handles

            pending = {c: fire(c) for c in range(min(3, nchunk))}
            zero = jnp.zeros((LANES,), jnp.float32)
            for c in range(nchunk):
                for cp in pending.pop(c):
                    cp.wait()
                par = c % 3
                first = c == 0

                def s_body(s, carry, par=par, first=first):
                    def rg_body(rg, lh):
                        lo_acc, hi_acc = lh
                        for u in range(4):
                            r = rg * 4 + u
                            lo = hi = None
                            for gref in (g0, g1, g2):
                                w = gref[par, r, pl.ds(s * LANES, LANES)]
                                a = lax.bitcast_convert_type(w << 16, jnp.float32)
                                b_ = lax.bitcast_convert_type(w, jnp.float32)
                                lo = a if lo is None else lo + a
                                hi = b_ if hi is None else hi + b_
                            lo_acc = lo_acc + 1.0 / (1.0 + jnp.exp(-lo))
                            hi_acc = hi_acc + 1.0 / (1.0 + jnp.exp(-hi))
                        return (lo_acc, hi_acc)

                    lo_acc, hi_acc = lax.fori_loop(0, C // 4, rg_body, (zero, zero))
                    dlo = pl.ds(s * LANES, LANES)
                    dhi = pl.ds(s * LANES + DC // 2, LANES)
                    if first:
                        accb[dlo] = lo_acc
                        accb[dhi] = hi_acc
                    else:
                        accb[dlo] = accb[dlo] + lo_acc
                        accb[dhi] = accb[dhi] + hi_acc
                    return carry

                lax.fori_loop(0, nsl, s_body, 0)
                if c + 3 < nchunk:
                    pending[c + 3] = fire(c + 3)
            pltpu.sync_copy(accb, cv_hbm.at[row])
            return carry

        lax.fori_loop(0, b_per_w, b_body, 0)

    return sc_combine


def _lse_body(cv_ref, w3_ref, b3_ref, m_ref, s_ref, *, vtar):
    j = pl.program_id(0)
    S = jnp.dot(cv_ref[...], w3_ref[...], preferred_element_type=jnp.float32) + b3_ref[...]
    col = j * V_TILE_A + lax.broadcasted_iota(jnp.int32, S.shape, 1)
    S = jnp.where(col < vtar, S, -1e30)
    neg = jnp.full_like(m_ref[...], -1e30)
    m_prev = jnp.where(j == 0, neg, m_ref[...])
    s_prev = jnp.where(j == 0, jnp.zeros_like(s_ref[...]), s_ref[...])
    m_cur = jnp.max(S, axis=1, keepdims=True)
    m_new = jnp.maximum(m_prev, m_cur)
    s_new = s_prev * jnp.exp(m_prev - m_new) + jnp.sum(jnp.exp(S - m_new), axis=1, keepdims=True)
    m_ref[...] = m_new
    s_ref[...] = s_new


def _soft_body(cv_ref, w3_ref, b3_ref, m_ref, s_ref, out_ref):
    S = jnp.dot(cv_ref[...], w3_ref[...], preferred_element_type=jnp.float32) + b3_ref[...]
    out_ref[...] = jnp.exp(S - m_ref[...]) * (1.0 / s_ref[...])


def _softmax_logits(cv, W3, b3):
    B, DC = cv.shape
    VTAR = W3.shape[1]
    grid = (pl.cdiv(VTAR, V_TILE),)
    b3r = b3.reshape(1, VTAR)
    m, s = pl.pallas_call(
        functools.partial(_lse_body, vtar=VTAR),
        grid=(pl.cdiv(VTAR, V_TILE_A),),
        in_specs=[
            pl.BlockSpec((B, DC), lambda j: (0, 0)),
            pl.BlockSpec((DC, V_TILE_A), lambda j: (0, j)),
            pl.BlockSpec((1, V_TILE_A), lambda j: (0, j)),
        ],
        out_specs=[
            pl.BlockSpec((B, 1), lambda j: (0, 0)),
            pl.BlockSpec((B, 1), lambda j: (0, 0)),
        ],
        out_shape=[
            jax.ShapeDtypeStruct((B, 1), jnp.float32),
            jax.ShapeDtypeStruct((B, 1), jnp.float32),
        ],
    )(cv, W3, b3r)
    return pl.pallas_call(
        _soft_body,
        grid=grid,
        in_specs=[
            pl.BlockSpec((B, DC), lambda j: (0, 0)),
            pl.BlockSpec((DC, V_TILE), lambda j: (0, j)),
            pl.BlockSpec((1, V_TILE), lambda j: (0, j)),
            pl.BlockSpec((B, 1), lambda j: (0, 0)),
            pl.BlockSpec((B, 1), lambda j: (0, 0)),
        ],
        out_specs=pl.BlockSpec((B, V_TILE), lambda j: (0, j)),
        out_shape=jax.ShapeDtypeStruct((B, VTAR), jnp.float32),
    )(cv, W3, b3r, m, s)


def kernel(source_tokens, path_indices, target_tokens, token_table, path_table,
           W1, b1, W2, b2, W3, b3):
    B, L = source_tokens.shape
    DC = W1.shape[0]
    tt1, tt2, pp = _project_tables(token_table, path_table, W1, b1)
    idx = jnp.stack([source_tokens, path_indices, target_tokens], axis=0)
    idx = idx.reshape(3, B, L // C, C).transpose(1, 2, 0, 3).reshape(B * (L // C), 3 * C)
    cv = _make_sc_combine(B, L, DC)(idx, tt1, pp, tt2)
    return _softmax_logits(cv, W3, b3)


# speedup vs baseline: 1.0143x; 1.0143x over previous
import functools

import jax
import jax.numpy as jnp
from jax import lax
from jax.experimental import pallas as pl
from jax.experimental.pallas import tpu as pltpu
from jax.experimental.pallas import tpu_sc as plsc

LANES = 16
NC, NS = 2, 16
NW = NC * NS
C = 40
ROW_TILE = 4096
V_TILE = 4096
V_TILE_A = 6144


PACKW = 256


def _pack_halves(y):
    h = y.shape[1] // 2
    lo = lax.bitcast_convert_type(y[:, :h].astype(jnp.bfloat16), jnp.uint16)
    hi = lax.bitcast_convert_type(y[:, h:].astype(jnp.bfloat16), jnp.uint16)
    word = lax.bitcast_convert_type(
        lo.astype(jnp.uint32) | (hi.astype(jnp.uint32) << 16), jnp.int32)
    pad = jnp.zeros((y.shape[0], PACKW - h), jnp.int32)
    return jnp.concatenate([word, pad], axis=1)


def _proj_body(tok_ref, pth_ref, w1a_ref, w1b_ref, w1c_ref, b1_ref,
               tt1_ref, tt2_ref, pp_ref):
    t = tok_ref[...]
    tt1_ref[...] = _pack_halves(
        jnp.dot(t, w1a_ref[...], preferred_element_type=jnp.float32) + b1_ref[...])
    tt2_ref[...] = _pack_halves(
        jnp.dot(t, w1c_ref[...], preferred_element_type=jnp.float32))
    pp_ref[...] = _pack_halves(
        jnp.dot(pth_ref[...], w1b_ref[...], preferred_element_type=jnp.float32))


def _project_tables(token_table, path_table, W1, b1):
    VT, DT = token_table.shape
    VP, DP = path_table.shape
    DC = W1.shape[0]
    W1a, W1b, W1c = W1[:DT], W1[DT:DT + DP], W1[DT + DP:]
    grid = (pl.cdiv(max(VT, VP), ROW_TILE),)
    return pl.pallas_call(
        _proj_body,
        grid=grid,
        in_specs=[
            pl.BlockSpec((ROW_TILE, DT), lambda i: (i, 0)),
            pl.BlockSpec((ROW_TILE, DP), lambda i: (i, 0)),
            pl.BlockSpec((DT, DC), lambda i: (0, 0)),
            pl.BlockSpec((DP, DC), lambda i: (0, 0)),
            pl.BlockSpec((DT, DC), lambda i: (0, 0)),
            pl.BlockSpec((1, DC), lambda i: (0, 0)),
        ],
        out_specs=[
            pl.BlockSpec((ROW_TILE, PACKW), lambda i: (i, 0)),
            pl.BlockSpec((ROW_TILE, PACKW), lambda i: (i, 0)),
            pl.BlockSpec((ROW_TILE, PACKW), lambda i: (i, 0)),
        ],
        out_shape=[
            jax.ShapeDtypeStruct((VT, PACKW), jnp.int32),
            jax.ShapeDtypeStruct((VT, PACKW), jnp.int32),
            jax.ShapeDtypeStruct((VP, PACKW), jnp.int32),
        ],
    )(token_table, path_table, W1a, W1b, W1c, b1.reshape(1, DC))


def _make_sc_combine(B, L, DC):
    nchunk = L // C
    b_per_w = B // NW
    idx_per_chunk = 3 * C
    DH = DC // 2
    nsl = DH // LANES
    mesh = plsc.VectorSubcoreMesh(core_axis_name="c", subcore_axis_name="s")

    @functools.partial(
        pl.kernel,
        out_type=jax.ShapeDtypeStruct((B, DC), jnp.float32),
        mesh=mesh,
        scratch_types=[
            pltpu.VMEM((b_per_w * nchunk, idx_per_chunk), jnp.int32),
            pltpu.VMEM((2, C, PACKW), jnp.int32),
            pltpu.VMEM((2, C, PACKW), jnp.int32),
            pltpu.VMEM((2, C, PACKW), jnp.int32),
            pltpu.VMEM((DC,), jnp.float32),
            pltpu.SemaphoreType.DMA,
            pltpu.SemaphoreType.DMA,
        ],
    )
    def sc_combine(idx_hbm, tt1_hbm, pp_hbm, tt2_hbm, cv_hbm,
                   idxall, g0, g1, g2, accb, sem0, sem1):
        wid = lax.axis_index("s") * NC + lax.axis_index("c")
        tables = (tt1_hbm, pp_hbm, tt2_hbm)
        bufs = (g0, g1, g2)
        sems = (sem0, sem1)
        nrows = b_per_w * nchunk
        pltpu.sync_copy(idx_hbm.at[pl.ds(wid * nrows, nrows)], idxall)

        def b_body(b, carry):
            row = wid * b_per_w + b

            def fire(c):
                par = c % 2
                handles = []
                for k in range(3):
                    cp = pltpu.make_async_copy(
                        tables[k].at[idxall.at[b * nchunk + c, pl.ds(k * C, C)]],
                        bufs[k].at[par], sems[par])
                    cp.start()
                    handles.append(cp)
                return handles

            pending = {c: fire(c) for c in range(min(2, nchunk))}
            zero = jnp.zeros((LANES,), jnp.float32)
            for c in range(nchunk):
                for cp in pending.pop(c):
                    cp.wait()
                par = c % 2
                first = c == 0

                def s_body(s, carry, par=par, first=first):
                    def rg_body(rg, lh):
                        lo_acc, hi_acc = lh
                        for u in range(4):
                            r = rg * 4 + u
                            lo = hi = None
                            for gref in (g0, g1, g2):
                                w = gref[par, r, pl.ds(s * LANES, LANES)]
                                a = lax.bitcast_convert_type(w << 16, jnp.float32)
                                b_ = lax.bitcast_convert_type(w, jnp.float32)
                                lo = a if lo is None else lo + a
                                hi = b_ if hi is None else hi + b_
                            lo_acc = lo_acc + 1.0 / (1.0 + jnp.exp(-lo))
                            hi_acc = hi_acc + 1.0 / (1.0 + jnp.exp(-hi))
                        return (lo_acc, hi_acc)

                    lo_acc, hi_acc = lax.fori_loop(0, C // 4, rg_body, (zero, zero))
                    dlo = pl.ds(s * LANES, LANES)
                    dhi = pl.ds(s * LANES + DC // 2, LANES)
                    if first:
                        accb[dlo] = lo_acc
                        accb[dhi] = hi_acc
                    else:
                        accb[dlo] = accb[dlo] + lo_acc
                        accb[dhi] = accb[dhi] + hi_acc
                    return carry

                lax.fori_loop(0, nsl, s_body, 0)
                if c + 2 < nchunk:
                    pending[c + 2] = fire(c + 2)
            pltpu.sync_copy(accb, cv_hbm.at[row])
            return carry

        lax.fori_loop(0, b_per_w, b_body, 0)

    return sc_combine


def _lse_body(cv_ref, w3_ref, b3_ref, m_ref, s_ref, *, vtar):
    j = pl.program_id(0)
    S = jnp.dot(cv_ref[...], w3_ref[...], preferred_element_type=jnp.float32) + b3_ref[...]
    col = j * V_TILE_A + lax.broadcasted_iota(jnp.int32, S.shape, 1)
    S = jnp.where(col < vtar, S, -1e30)
    neg = jnp.full_like(m_ref[...], -1e30)
    m_prev = jnp.where(j == 0, neg, m_ref[...])
    s_prev = jnp.where(j == 0, jnp.zeros_like(s_ref[...]), s_ref[...])
    m_cur = jnp.max(S, axis=1, keepdims=True)
    m_new = jnp.maximum(m_prev, m_cur)
    s_new = s_prev * jnp.exp(m_prev - m_new) + jnp.sum(jnp.exp(S - m_new), axis=1, keepdims=True)
    m_ref[...] = m_new
    s_ref[...] = s_new


def _soft_body(cv_ref, w3_ref, b3_ref, m_ref, s_ref, out_ref):
    S = jnp.dot(cv_ref[...], w3_ref[...], preferred_element_type=jnp.float32) + b3_ref[...]
    out_ref[...] = jnp.exp(S - m_ref[...]) * (1.0 / s_ref[...])


def _softmax_logits(cv, W3, b3):
    B, DC = cv.shape
    VTAR = W3.shape[1]
    grid = (pl.cdiv(VTAR, V_TILE),)
    b3r = b3.reshape(1, VTAR)
    m, s = pl.pallas_call(
        functools.partial(_lse_body, vtar=VTAR),
        grid=(pl.cdiv(VTAR, V_TILE_A),),
        in_specs=[
            pl.BlockSpec((B, DC), lambda j: (0, 0)),
            pl.BlockSpec((DC, V_TILE_A), lambda j: (0, j)),
            pl.BlockSpec((1, V_TILE_A), lambda j: (0, j)),
        ],
        out_specs=[
            pl.BlockSpec((B, 1), lambda j: (0, 0)),
            pl.BlockSpec((B, 1), lambda j: (0, 0)),
        ],
        out_shape=[
            jax.ShapeDtypeStruct((B, 1), jnp.float32),
            jax.ShapeDtypeStruct((B, 1), jnp.float32),
        ],
    )(cv, W3, b3r)
    return pl.pallas_call(
        _soft_body,
        grid=grid,
        in_specs=[
            pl.BlockSpec((B, DC), lambda j: (0, 0)),
            pl.BlockSpec((DC, V_TILE), lambda j: (0, j)),
            pl.BlockSpec((1, V_TILE), lambda j: (0, j)),
            pl.BlockSpec((B, 1), lambda j: (0, 0)),
            pl.BlockSpec((B, 1), lambda j: (0, 0)),
        ],
        out_specs=pl.BlockSpec((B, V_TILE), lambda j: (0, j)),
        out_shape=jax.ShapeDtypeStruct((B, VTAR), jnp.float32),
    )(cv, W3, b3r, m, s)


def kernel(source_tokens, path_indices, target_tokens, token_table, path_table,
           W1, b1, W2, b2, W3, b3):
    B, L = source_tokens.shape
    DC = W1.shape[0]
    tt1, tt2, pp = _project_tables(token_table, path_table, W1, b1)
    idx = jnp.stack([source_tokens, path_indices, target_tokens], axis=0)
    idx = idx.reshape(3, B, L // C, C).transpose(1, 2, 0, 3).reshape(B * (L // C), 3 * C)
    cv = _make_sc_combine(B, L, DC)(idx, tt1, pp, tt2)
    return _softmax_logits(cv, W3, b3)
